# Initial kernel scaffold; baseline (speedup 1.0000x reference)
import jax
import jax.numpy as jnp
from jax.experimental import pallas as pl


def kernel(big_batch_positions, big_batched_adjacency_pruned, ego_mask_batch, W1, b1, W2, b2, se_w1, se_b1, se_w2, se_b2, out_w, out_b):
    raise NotImplementedError("write your pallas kernel here")


# speedup vs baseline: 2.7969x; 2.7969x over previous
import functools

import jax
import jax.numpy as jnp
from jax.experimental import pallas as pl

T = 8
B = 4
N = 256
BN = B * N
D_IN = 128
HID = 128
D_OUT = 64
SQ = T // 2

_F32 = jnp.float32


def _gcn_step(m_ref, x_ref, a_ref, w1_ref, b1_ref, w2_ref, b2_ref, out_ref):
    m = m_ref[0, 0, :]
    a = a_ref[0]
    x = x_ref[0]

    atm = jax.lax.dot_general(
        m.reshape(1, BN), a,
        (((1,), (0,)), ((), ())),
        preferred_element_type=_F32,
    )[0]
    dinv = m * jax.lax.rsqrt(atm + 1.0)
    d2 = dinv * dinv

    def conv(h, b):
        v = dinv[:, None] * h
        u = jax.lax.dot_general(
            a, v, (((0,), (0,)), ((), ())),
            preferred_element_type=_F32,
        )
        return dinv[:, None] * u + d2[:, None] * h + b

    h = jnp.dot(x, w1_ref[...], preferred_element_type=_F32)
    h1 = jnp.maximum(conv(h, b1_ref[0]), 0.0)
    hb = jnp.dot(h1, w2_ref[...], preferred_element_type=_F32)
    h2 = conv(hb, b2_ref[0])
    out_ref[0] = m[:, None] * h2


def _se_combine(z_ref, m_ref, sw1_ref, sb1_ref, sw2_ref, sb2_ref,
                ow_ref, ob_ref, out_ref):
    z = z_ref[...]
    m = m_ref[...]
    n = jnp.sum(m, axis=(1, 2))
    csum = jnp.sum(z, axis=(1, 2))
    c = jnp.where(n > 0, csum / (n * HID), 0.0)
    s1 = jnp.maximum(jnp.sum(c[:, None] * sw1_ref[...], axis=0) + sb1_ref[0], 0.0)
    s = jax.nn.sigmoid(jnp.sum(s1[:, None] * sw2_ref[...], axis=0) + sb2_ref[0])
    zh = jnp.sum(s[:, None, None] * z, axis=0)
    out_ref[...] = (
        jnp.dot(zh, ow_ref[...], preferred_element_type=_F32) + ob_ref[0]
    )


@functools.partial(jax.jit, static_argnames=())
def kernel(big_batch_positions, big_batched_adjacency_pruned, ego_mask_batch,
           W1, b1, W2, b2, se_w1, se_b1, se_w2, se_b2, out_w, out_b):
    x = big_batch_positions
    A = big_batched_adjacency_pruned
    m = jnp.transpose(ego_mask_batch, (1, 0, 2)).reshape(T, 1, BN).astype(_F32)

    z = pl.pallas_call(
        _gcn_step,
        grid=(T,),
        in_specs=[
            pl.BlockSpec((1, 1, BN), lambda t: (t, 0, 0)),
            pl.BlockSpec((1, BN, D_IN), lambda t: (t, 0, 0)),
            pl.BlockSpec((1, BN, BN), lambda t: (t, 0, 0)),
            pl.BlockSpec((D_IN, HID), lambda t: (0, 0)),
            pl.BlockSpec((1, HID), lambda t: (0, 0)),
            pl.BlockSpec((HID, HID), lambda t: (0, 0)),
            pl.BlockSpec((1, HID), lambda t: (0, 0)),
        ],
        out_specs=pl.BlockSpec((1, BN, HID), lambda t: (t, 0, 0)),
        out_shape=jax.ShapeDtypeStruct((T, BN, HID), _F32),
    )(m, x, A, W1, b1.reshape(1, HID), W2, b2.reshape(1, HID))

    out = pl.pallas_call(
        _se_combine,
        in_specs=[
            pl.BlockSpec((T, BN, HID), lambda: (0, 0, 0)),
            pl.BlockSpec((T, 1, BN), lambda: (0, 0, 0)),
            pl.BlockSpec((T, SQ), lambda: (0, 0)),
            pl.BlockSpec((1, SQ), lambda: (0, 0)),
            pl.BlockSpec((SQ, T), lambda: (0, 0)),
            pl.BlockSpec((1, T), lambda: (0, 0)),
            pl.BlockSpec((HID, D_OUT), lambda: (0, 0)),
            pl.BlockSpec((1, D_OUT), lambda: (0, 0)),
        ],
        out_specs=pl.BlockSpec((BN, D_OUT), lambda: (0, 0)),
        out_shape=jax.ShapeDtypeStruct((BN, D_OUT), _F32),
    )(z, m, se_w1, se_b1.reshape(1, SQ), se_w2, se_b2.reshape(1, T),
      out_w, out_b.reshape(1, D_OUT))

    out = out.reshape(B, N, D_OUT)
    return jnp.broadcast_to(out[:, :, None, :], (B, N, T, D_OUT))
